# baseline (speedup 1.0000x reference)
import functools

import jax
import jax.numpy as jnp
from jax import lax
from jax.experimental import pallas as pl
from jax.experimental.pallas import tpu as pltpu
from jax.experimental.pallas import tpu_sc as plsc

M, K, N, E, TOPK = 2048, 1024, 1024, 8, 2
P = M * TOPK
BM = 256
NB = P // BM + E
PT = NB * BM

NC, NS = 2, 16
NW = NC * NS

RPW = PT // NW
ACH = 96
NACH = RPW // ACH
KW = K // 2

TPW = M // NW
TCH = 16
NTCH = TPW // TCH


def _sc_mesh():
    return plsc.VectorSubcoreMesh(
        core_axis_name="c", subcore_axis_name="s", num_cores=NC, num_subcores=NS
    )


def _wid():
    return lax.axis_index("s") * NC + lax.axis_index("c")


def _gather_body(src_hbm, a1_hbm, out_hbm, idx_v,
                 buf0, buf1, gsem0, gsem1, wsem0, wsem1):
    base = _wid() * RPW
    pltpu.sync_copy(src_hbm.at[pl.ds(base, RPW)], idx_v)
    g0 = pltpu.async_copy(a1_hbm.at[idx_v.at[pl.ds(0, ACH)]], buf0, gsem0)
    g1 = pltpu.async_copy(a1_hbm.at[idx_v.at[pl.ds(ACH, ACH)]], buf1, gsem1)
    g0.wait()
    w0 = pltpu.async_copy(buf0, out_hbm.at[pl.ds(base, ACH)], wsem0)
    g1.wait()
    w1 = pltpu.async_copy(buf1, out_hbm.at[pl.ds(base + ACH, ACH)], wsem1)
    w0.wait()
    w1.wait()


def _sc_gather(src, a1_words):
    k = functools.partial(
        pl.kernel,
        out_type=jax.ShapeDtypeStruct((PT, KW), jnp.int32),
        mesh=_sc_mesh(),
        scratch_types=[
            pltpu.VMEM((RPW,), jnp.int32),
            pltpu.VMEM((ACH, KW), jnp.int32),
            pltpu.VMEM((ACH, KW), jnp.int32),
            pltpu.SemaphoreType.DMA,
            pltpu.SemaphoreType.DMA,
            pltpu.SemaphoreType.DMA,
            pltpu.SemaphoreType.DMA,
        ],
    )(_gather_body)
    return k(src, a1_words)


def _combine_body(dest_hbm, tw0_hbm, tw1_hbm, osort_hbm, out_hbm,
                  idx_v, tw0_v, tw1_v, rbuf0, rbuf1, obuf, sem0, sem1):
    w = _wid()
    tbase = w * TPW
    pltpu.sync_copy(dest_hbm.at[pl.ds(tbase * TOPK, TPW * TOPK)], idx_v)
    pltpu.sync_copy(tw0_hbm.at[pl.ds(tbase, TPW)], tw0_v)
    pltpu.sync_copy(tw1_hbm.at[pl.ds(tbase, TPW)], tw1_v)
    bufs = (rbuf0, rbuf1)
    sems = (sem0, sem1)
    copies = [None, None]
    copies[0] = pltpu.async_copy(
        osort_hbm.at[idx_v.at[pl.ds(0, TCH * TOPK)]], rbuf0, sem0
    )
    for c in range(NTCH):
        nxt = (c + 1) % 2
        if c + 1 < NTCH:
            copies[nxt] = pltpu.async_copy(
                osort_hbm.at[idx_v.at[pl.ds((c + 1) * TCH * TOPK, TCH * TOPK)]],
                bufs[nxt], sems[nxt],
            )
        copies[c % 2].wait()
        rbuf = bufs[c % 2]
        w0c = tw0_v[pl.ds(c * TCH, 16)]
        w1c = tw1_v[pl.ds(c * TCH, 16)]
        for j in range(TCH):
            w0 = w0c[j]
            w1 = w1c[j]

            def lane_body(l, _, j=j, w0=w0, w1=w1):
                r0 = rbuf[2 * j, pl.ds(l * 16, 16)]
                r1 = rbuf[2 * j + 1, pl.ds(l * 16, 16)]
                obuf[j, pl.ds(l * 16, 16)] = w0 * r0 + w1 * r1
                return 0

            lax.fori_loop(0, K // 16, lane_body, 0, unroll=8)
        pltpu.sync_copy(obuf, out_hbm.at[pl.ds(tbase + c * TCH, TCH)])


def _sc_combine(dest, tw0, tw1, o_sorted):
    k = functools.partial(
        pl.kernel,
        out_type=jax.ShapeDtypeStruct((M, K), jnp.float32),
        mesh=_sc_mesh(),
        scratch_types=[
            pltpu.VMEM((TPW * TOPK,), jnp.int32),
            pltpu.VMEM((TPW,), jnp.float32),
            pltpu.VMEM((TPW,), jnp.float32),
            pltpu.VMEM((TCH * TOPK, K), jnp.float32),
            pltpu.VMEM((TCH * TOPK, K), jnp.float32),
            pltpu.VMEM((TCH, K), jnp.float32),
            pltpu.SemaphoreType.DMA,
            pltpu.SemaphoreType.DMA,
        ],
    )(_combine_body)
    return k(dest, tw0, tw1, o_sorted)


def _gemm_body(blk_e_ref, nb_ref, a_ref, w1_ref, w2_ref, o_ref):
    b = pl.program_id(0)

    @pl.when(b < nb_ref[0])
    def _():
        a = a_ref[...].astype(jnp.float32)
        h = lax.dot_general(a, w1_ref[0], (((1,), (1,)), ((), ())),
                            preferred_element_type=jnp.float32)
        gate = h[:, :N]
        up = h[:, N:]
        act = (gate * jax.nn.sigmoid(gate)) * up
        o_ref[...] = lax.dot_general(act, w2_ref[0], (((1,), (1,)), ((), ())),
                                     preferred_element_type=jnp.float32)


def _grouped_gemm(blk_e, nb_real, a_sorted, w1, w2):
    grid_spec = pltpu.PrefetchScalarGridSpec(
        num_scalar_prefetch=2,
        grid=(NB,),
        in_specs=[
            pl.BlockSpec((BM, K), lambda b, se, sn: (b, 0)),
            pl.BlockSpec((1, 2 * N, K), lambda b, se, sn: (se[b], 0, 0)),
            pl.BlockSpec((1, K, N), lambda b, se, sn: (se[b], 0, 0)),
        ],
        out_specs=pl.BlockSpec((BM, K), lambda b, se, sn: (b, 0)),
    )
    return pl.pallas_call(
        _gemm_body,
        grid_spec=grid_spec,
        out_shape=jax.ShapeDtypeStruct((PT, K), jnp.float32),
        compiler_params=pltpu.CompilerParams(
            dimension_semantics=("arbitrary",),
        ),
    )(blk_e, nb_real, a_sorted, w1, w2)


@jax.jit
def kernel(a1, w1, w2, topk_weights, topk_ids):
    ids = topk_ids.astype(jnp.int32)
    e_flat = ids.reshape(-1)
    onehot = (e_flat[:, None] == jnp.arange(E, dtype=jnp.int32)[None, :]
              ).astype(jnp.int32)
    incl = jnp.cumsum(onehot, axis=0)
    counts = incl[-1]
    rank = jnp.take_along_axis(incl - onehot, e_flat[:, None], axis=1)[:, 0]
    padded = ((counts + BM - 1) // BM) * BM
    ends = jnp.cumsum(padded)
    base = ends - padded
    dest = (base[e_flat] + rank).astype(jnp.int32)
    tok = jnp.arange(P, dtype=jnp.int32) // TOPK
    src = jnp.zeros((PT,), jnp.int32).at[dest].set(tok)
    blk_e = jnp.searchsorted(
        ends, jnp.arange(NB, dtype=jnp.int32) * BM, side="right"
    ).astype(jnp.int32)
    blk_e = jnp.minimum(blk_e, E - 1)
    nb_real = (ends[-1] // BM).astype(jnp.int32).reshape((1,))

    a1_words = lax.bitcast_convert_type(
        a1.astype(jnp.bfloat16).reshape(M, KW, 2), jnp.int32)
    a_sorted_w = _sc_gather(src, a1_words)
    a_sorted = lax.bitcast_convert_type(
        a_sorted_w, jnp.bfloat16).reshape(PT, K)
    o_sorted = _grouped_gemm(blk_e, nb_real, a_sorted, w1, w2)
    tw0 = topk_weights[:, 0]
    tw1 = topk_weights[:, 1]
    out = _sc_combine(dest, tw0, tw1, o_sorted)
    return out


# speedup vs baseline: 1.3360x; 1.3360x over previous
import functools

import jax
import jax.numpy as jnp
from jax import lax
from jax.experimental import pallas as pl
from jax.experimental.pallas import tpu as pltpu
from jax.experimental.pallas import tpu_sc as plsc

M, K, N, E, TOPK = 2048, 1024, 1024, 8, 2
P = M * TOPK
BM = 256
NB = P // BM + E
PT = NB * BM

NC, NS = 2, 16
NW = NC * NS

RPW = PT // NW
ACH = 48
NACH = RPW // ACH
SL = 8

TPW = M // NW
TCH = 16
NTCH = TPW // TCH


def _sc_mesh():
    return plsc.VectorSubcoreMesh(
        core_axis_name="c", subcore_axis_name="s", num_cores=NC, num_subcores=NS
    )


def _wid():
    return lax.axis_index("s") * NC + lax.axis_index("c")


def _gather_body(src_hbm, a1_hbm, out_hbm, idx_v,
                 buf0, buf1, gsem0, gsem1, wsem0, wsem1):
    base = _wid() * RPW
    pltpu.sync_copy(src_hbm.at[pl.ds(base, RPW)], idx_v)
    bufs = (buf0, buf1)
    gsems = (gsem0, gsem1)
    wsems = (wsem0, wsem1)
    gcp = [None, None]
    wcp = [None, None]
    gcp[0] = pltpu.async_copy(a1_hbm.at[idx_v.at[pl.ds(0, ACH)]], buf0, gsem0)
    for c in range(NACH):
        k = c % 2
        nk = (c + 1) % 2
        if c + 1 < NACH:
            if wcp[nk] is not None:
                wcp[nk].wait()
            gcp[nk] = pltpu.async_copy(
                a1_hbm.at[idx_v.at[pl.ds((c + 1) * ACH, ACH)]], bufs[nk],
                gsems[nk])
        gcp[k].wait()
        wcp[k] = pltpu.async_copy(
            bufs[k], out_hbm.at[pl.ds(base + c * ACH, ACH)], wsems[k])
    wcp[0].wait()
    wcp[1].wait()


def _sc_gather(src, a1_3d):
    k = functools.partial(
        pl.kernel,
        out_type=jax.ShapeDtypeStruct((PT, SL, 128), jnp.float32),
        mesh=_sc_mesh(),
        scratch_types=[
            pltpu.VMEM((RPW,), jnp.int32),
            pltpu.VMEM((ACH, SL, 128), jnp.float32),
            pltpu.VMEM((ACH, SL, 128), jnp.float32),
            pltpu.SemaphoreType.DMA,
            pltpu.SemaphoreType.DMA,
            pltpu.SemaphoreType.DMA,
            pltpu.SemaphoreType.DMA,
        ],
    )(_gather_body)
    return k(src, a1_3d)


def _combine_body(dest_hbm, tw0_hbm, tw1_hbm, osort_hbm, out_hbm,
                  idx_v, tw0_v, tw1_v, rbuf0, rbuf1, obuf, sem0, sem1):
    w = _wid()
    tbase = w * TPW
    pltpu.sync_copy(dest_hbm.at[pl.ds(tbase * TOPK, TPW * TOPK)], idx_v)
    pltpu.sync_copy(tw0_hbm.at[pl.ds(tbase, TPW)], tw0_v)
    pltpu.sync_copy(tw1_hbm.at[pl.ds(tbase, TPW)], tw1_v)
    bufs = (rbuf0, rbuf1)
    sems = (sem0, sem1)
    copies = [None, None]
    copies[0] = pltpu.async_copy(
        osort_hbm.at[idx_v.at[pl.ds(0, TCH * TOPK)]], rbuf0, sem0
    )
    for c in range(NTCH):
        nxt = (c + 1) % 2
        if c + 1 < NTCH:
            copies[nxt] = pltpu.async_copy(
                osort_hbm.at[idx_v.at[pl.ds((c + 1) * TCH * TOPK, TCH * TOPK)]],
                bufs[nxt], sems[nxt],
            )
        copies[c % 2].wait()
        rbuf = bufs[c % 2]
        w0c = tw0_v[pl.ds(c * TCH, 16)]
        w1c = tw1_v[pl.ds(c * TCH, 16)]
        for j in range(TCH):
            w0 = w0c[j]
            w1 = w1c[j]

            def sub_body(s, _, j=j, w0=w0, w1=w1):
                def lane_body(l, _2, s=s, j=j, w0=w0, w1=w1):
                    r0 = rbuf[2 * j, s, pl.ds(l * 16, 16)]
                    r1 = rbuf[2 * j + 1, s, pl.ds(l * 16, 16)]
                    obuf[j, s, pl.ds(l * 16, 16)] = w0 * r0 + w1 * r1
                    return 0

                lax.fori_loop(0, 128 // 16, lane_body, 0, unroll=4)
                return 0

            lax.fori_loop(0, SL, sub_body, 0)
        pltpu.sync_copy(obuf, out_hbm.at[pl.ds(tbase + c * TCH, TCH)])


def _sc_combine(dest, tw0, tw1, o_sorted):
    k = functools.partial(
        pl.kernel,
        out_type=jax.ShapeDtypeStruct((M, SL, 128), jnp.float32),
        mesh=_sc_mesh(),
        scratch_types=[
            pltpu.VMEM((TPW * TOPK,), jnp.int32),
            pltpu.VMEM((TPW,), jnp.float32),
            pltpu.VMEM((TPW,), jnp.float32),
            pltpu.VMEM((TCH * TOPK, SL, 128), jnp.float32),
            pltpu.VMEM((TCH * TOPK, SL, 128), jnp.float32),
            pltpu.VMEM((TCH, SL, 128), jnp.float32),
            pltpu.SemaphoreType.DMA,
            pltpu.SemaphoreType.DMA,
        ],
    )(_combine_body)
    return k(dest, tw0, tw1, o_sorted)


def _gemm_body(blk_e_ref, nb_ref, a_ref, w1_ref, w2_ref, o_ref):
    b = pl.program_id(0)

    @pl.when(b < nb_ref[0])
    def _():
        a = a_ref[...]
        h = lax.dot_general(a, w1_ref[0], (((1,), (1,)), ((), ())),
                            preferred_element_type=jnp.float32)
        gate = h[:, :N]
        up = h[:, N:]
        act = (gate * jax.nn.sigmoid(gate)) * up
        o_ref[...] = lax.dot_general(act, w2_ref[0], (((1,), (1,)), ((), ())),
                                     preferred_element_type=jnp.float32)


def _grouped_gemm(blk_e, nb_real, a_sorted, w1, w2):
    grid_spec = pltpu.PrefetchScalarGridSpec(
        num_scalar_prefetch=2,
        grid=(NB,),
        in_specs=[
            pl.BlockSpec((BM, K), lambda b, se, sn: (b, 0)),
            pl.BlockSpec((1, 2 * N, K), lambda b, se, sn: (se[b], 0, 0)),
            pl.BlockSpec((1, K, N), lambda b, se, sn: (se[b], 0, 0)),
        ],
        out_specs=pl.BlockSpec((BM, K), lambda b, se, sn: (b, 0)),
    )
    return pl.pallas_call(
        _gemm_body,
        grid_spec=grid_spec,
        out_shape=jax.ShapeDtypeStruct((PT, K), jnp.float32),
        compiler_params=pltpu.CompilerParams(
            dimension_semantics=("arbitrary",),
        ),
    )(blk_e, nb_real, a_sorted, w1, w2)


@jax.jit
def kernel(a1, w1, w2, topk_weights, topk_ids):
    ids = topk_ids.astype(jnp.int32)
    e_flat = ids.reshape(-1)
    onehot = (e_flat[:, None] == jnp.arange(E, dtype=jnp.int32)[None, :]
              ).astype(jnp.int32)
    incl = jnp.cumsum(onehot, axis=0)
    counts = incl[-1]
    rank = jnp.take_along_axis(incl - onehot, e_flat[:, None], axis=1)[:, 0]
    padded = ((counts + BM - 1) // BM) * BM
    ends = jnp.cumsum(padded)
    base = ends - padded
    dest = (base[e_flat] + rank).astype(jnp.int32)
    tok = jnp.arange(P, dtype=jnp.int32) // TOPK
    src = jnp.zeros((PT,), jnp.int32).at[dest].set(tok)
    blk_e = jnp.searchsorted(
        ends, jnp.arange(NB, dtype=jnp.int32) * BM, side="right"
    ).astype(jnp.int32)
    blk_e = jnp.minimum(blk_e, E - 1)
    nb_real = (ends[-1] // BM).astype(jnp.int32).reshape((1,))

    a_sorted = _sc_gather(src, a1.reshape(M, SL, 128)).reshape(PT, K)
    o_sorted = _grouped_gemm(blk_e, nb_real, a_sorted, w1, w2)
    tw0 = topk_weights[:, 0]
    tw1 = topk_weights[:, 1]
    out = _sc_combine(dest, tw0, tw1, o_sorted.reshape(PT, SL, 128))
    return out.reshape(M, K)


# speedup vs baseline: 2.3837x; 1.7843x over previous
import functools

import jax
import jax.numpy as jnp
from jax import lax
from jax.experimental import pallas as pl
from jax.experimental.pallas import tpu as pltpu

M, K, N, E, TOPK = 2048, 1024, 1024, 8, 2
P = M * TOPK
BM = 256
NB = P // BM + E
PT = NB * BM
CM = 256


def _gemm_body(blk_e_ref, nb_ref, src_ref, a1_ref, w1_ref, w2_ref, o_ref):
    b = pl.program_id(0)

    @pl.when(b < nb_ref[0])
    def _():
        src_row = src_ref[0, 0, :]
        tok = lax.broadcasted_iota(jnp.int32, (BM, M), 1)
        sel = (tok == src_row[:, None]).astype(jnp.bfloat16)
        ag = jnp.dot(sel, a1_ref[...],
                     preferred_element_type=jnp.float32)
        h = lax.dot_general(ag, w1_ref[0], (((1,), (1,)), ((), ())),
                            preferred_element_type=jnp.float32)
        gate = h[:, :N]
        up = h[:, N:]
        act = (gate * jax.nn.sigmoid(gate)) * up
        o_ref[...] = lax.dot_general(
            act, w2_ref[0], (((1,), (1,)), ((), ())),
            preferred_element_type=jnp.float32).astype(jnp.bfloat16)

    @pl.when(b >= nb_ref[0])
    def _():
        o_ref[...] = jnp.zeros((BM, K), jnp.bfloat16)


def _grouped_gemm(blk_e, nb_real, src_b, a1_bf, w1, w2):
    grid_spec = pltpu.PrefetchScalarGridSpec(
        num_scalar_prefetch=2,
        grid=(NB,),
        in_specs=[
            pl.BlockSpec((1, 1, BM), lambda b, se, sn: (b, 0, 0)),
            pl.BlockSpec((M, K), lambda b, se, sn: (0, 0)),
            pl.BlockSpec((1, 2 * N, K), lambda b, se, sn: (se[b], 0, 0)),
            pl.BlockSpec((1, K, N), lambda b, se, sn: (se[b], 0, 0)),
        ],
        out_specs=pl.BlockSpec((BM, K), lambda b, se, sn: (b, 0)),
    )
    return pl.pallas_call(
        _gemm_body,
        grid_spec=grid_spec,
        out_shape=jax.ShapeDtypeStruct((PT, K), jnp.bfloat16),
        compiler_params=pltpu.CompilerParams(
            dimension_semantics=("arbitrary",),
        ),
    )(blk_e, nb_real, src_b, a1_bf, w1, w2)


def _combine_body(dest_ref, tw_ref, o_ref, out_ref):
    d0 = dest_ref[0, 0, :]
    d1 = dest_ref[0, 1, :]
    tw = tw_ref[...]
    slot = lax.broadcasted_iota(jnp.int32, (CM, PT), 1)
    w = (jnp.where(slot == d0[:, None], tw[:, 0:1], 0.0)
         + jnp.where(slot == d1[:, None], tw[:, 1:2], 0.0)
         ).astype(jnp.bfloat16)
    out_ref[...] = jnp.dot(w, o_ref[...],
                           preferred_element_type=jnp.float32)


def _combine(dest_b, topk_weights, o_sorted_bf):
    grid_spec = pltpu.PrefetchScalarGridSpec(
        num_scalar_prefetch=0,
        grid=(M // CM,),
        in_specs=[
            pl.BlockSpec((1, TOPK, CM), lambda c: (c, 0, 0)),
            pl.BlockSpec((CM, TOPK), lambda c: (c, 0)),
            pl.BlockSpec((PT, K), lambda c: (0, 0)),
        ],
        out_specs=pl.BlockSpec((CM, K), lambda c: (c, 0)),
    )
    return pl.pallas_call(
        _combine_body,
        grid_spec=grid_spec,
        out_shape=jax.ShapeDtypeStruct((M, K), jnp.float32),
        compiler_params=pltpu.CompilerParams(
            dimension_semantics=("arbitrary",),
        ),
    )(dest_b, topk_weights, o_sorted_bf)


@jax.jit
def kernel(a1, w1, w2, topk_weights, topk_ids):
    ids = topk_ids.astype(jnp.int32)
    e_flat = ids.reshape(-1)
    onehot = (e_flat[:, None] == jnp.arange(E, dtype=jnp.int32)[None, :]
              ).astype(jnp.int32)
    incl = jnp.cumsum(onehot, axis=0)
    counts = incl[-1]
    rank = jnp.take_along_axis(incl - onehot, e_flat[:, None], axis=1)[:, 0]
    padded = ((counts + BM - 1) // BM) * BM
    ends = jnp.cumsum(padded)
    base = ends - padded
    dest = (base[e_flat] + rank).astype(jnp.int32)
    tok = jnp.arange(P, dtype=jnp.int32) // TOPK
    src = jnp.full((PT,), M, jnp.int32).at[dest].set(tok)
    blk_e = jnp.searchsorted(
        ends, jnp.arange(NB, dtype=jnp.int32) * BM, side="right"
    ).astype(jnp.int32)
    blk_e = jnp.minimum(blk_e, E - 1)
    nb_real = (ends[-1] // BM).astype(jnp.int32).reshape((1,))

    a1_bf = a1.astype(jnp.bfloat16)
    o_sorted = _grouped_gemm(blk_e, nb_real, src.reshape(NB, 1, BM),
                             a1_bf, w1, w2)
    dest_b = (dest.reshape(M // CM, CM, TOPK)
              .transpose(0, 2, 1).reshape(M // CM, TOPK, CM))
    out = _combine(dest_b, topk_weights, o_sorted)
    return out


# speedup vs baseline: 3.1733x; 1.3312x over previous
import functools

import jax
import jax.numpy as jnp
from jax import lax
from jax.experimental import pallas as pl
from jax.experimental.pallas import tpu as pltpu

M, K, N, E, TOPK = 2048, 1024, 1024, 8, 2


def _moe_dense_body(a_ref, w1_ref, w2_ref, tw_ref, ids_ref, out_ref):
    e = pl.program_id(0)
    a = a_ref[...]
    w1e = w1_ref[0]
    w2e = w2_ref[0]
    h = lax.dot_general(a, w1e, (((1,), (1,)), ((), ())),
                        preferred_element_type=jnp.float32)
    gate = h[:, :N]
    up = h[:, N:]
    act = (gate * jax.nn.sigmoid(gate)) * up
    o = lax.dot_general(act, w2e, (((1,), (1,)), ((), ())),
                        preferred_element_type=jnp.float32)
    w = jnp.sum(tw_ref[...] * (ids_ref[...] == e).astype(jnp.float32),
                axis=1, keepdims=True)
    contrib = w * o

    @pl.when(e == 0)
    def _():
        out_ref[...] = contrib

    @pl.when(e > 0)
    def _():
        out_ref[...] += contrib


@jax.jit
def kernel(a1, w1, w2, topk_weights, topk_ids):
    ids = topk_ids.astype(jnp.int32)
    out = pl.pallas_call(
        _moe_dense_body,
        grid=(E,),
        in_specs=[
            pl.BlockSpec((M, K), lambda e: (0, 0)),
            pl.BlockSpec((1, 2 * N, K), lambda e: (e, 0, 0)),
            pl.BlockSpec((1, K, N), lambda e: (e, 0, 0)),
            pl.BlockSpec((M, TOPK), lambda e: (0, 0)),
            pl.BlockSpec((M, TOPK), lambda e: (0, 0)),
        ],
        out_specs=pl.BlockSpec((M, K), lambda e: (0, 0)),
        out_shape=jax.ShapeDtypeStruct((M, K), jnp.float32),
        compiler_params=pltpu.CompilerParams(
            dimension_semantics=("arbitrary",),
        ),
    )(a1, w1, w2, topk_weights, ids)
    return out


# speedup vs baseline: 3.2911x; 1.0371x over previous
import functools

import jax
import jax.numpy as jnp
from jax import lax
from jax.experimental import pallas as pl
from jax.experimental.pallas import tpu as pltpu

M, K, N, E, TOPK = 2048, 1024, 1024, 8, 2


def _moe_dense_body(a_ref, w1_ref, w2_ref, tw_ref, ids_ref, out_ref):
    e = pl.program_id(0)
    a = a_ref[...]
    w1e = w1_ref[0]
    w2e = w2_ref[0]
    h = lax.dot_general(a, w1e, (((1,), (1,)), ((), ())),
                        preferred_element_type=jnp.float32
                        ).astype(jnp.bfloat16)
    gate = h[:, :N]
    up = h[:, N:]
    act = (gate * jax.nn.sigmoid(gate)) * up
    o = lax.dot_general(act, w2e.astype(jnp.bfloat16),
                        (((1,), (1,)), ((), ())),
                        preferred_element_type=jnp.float32)
    w = jnp.sum(tw_ref[...] * (ids_ref[...] == e).astype(jnp.float32),
                axis=1, keepdims=True)
    contrib = w * o

    @pl.when(e == 0)
    def _():
        out_ref[...] = contrib

    @pl.when(e > 0)
    def _():
        out_ref[...] += contrib


@jax.jit
def kernel(a1, w1, w2, topk_weights, topk_ids):
    ids = topk_ids.astype(jnp.int32)
    out = pl.pallas_call(
        _moe_dense_body,
        grid=(E,),
        in_specs=[
            pl.BlockSpec((M, K), lambda e: (0, 0)),
            pl.BlockSpec((1, 2 * N, K), lambda e: (e, 0, 0)),
            pl.BlockSpec((1, K, N), lambda e: (e, 0, 0)),
            pl.BlockSpec((M, TOPK), lambda e: (0, 0)),
            pl.BlockSpec((M, TOPK), lambda e: (0, 0)),
        ],
        out_specs=pl.BlockSpec((M, K), lambda e: (0, 0)),
        out_shape=jax.ShapeDtypeStruct((M, K), jnp.float32),
        compiler_params=pltpu.CompilerParams(
            dimension_semantics=("arbitrary",),
        ),
    )(a1, w1, w2, topk_weights, ids)
    return out
